# baseline (speedup 1.0000x reference)
import functools

import jax
import jax.numpy as jnp
from jax import lax
from jax.experimental import pallas as pl
from jax.experimental.pallas import tpu as pltpu
from jax.experimental.pallas import tpu_sc as plsc

N = 10000
E = 320000
D = 128
H = 64
G = 64
C = 10
EPS = 1e-5

_NC = 2
_NS = 16
_NW = _NC * _NS
_K = 128
_NCHUNK = 80
_GRP = 4
_NGRP = _NCHUNK // _GRP
_EPW = _NCHUNK * _K
_EPAD = _NW * _EPW
_NPAD = 10240
_RPT = _NPAD // _NS

_mesh = plsc.VectorSubcoreMesh(
    core_axis_name="c", subcore_axis_name="s", num_cores=_NC, num_subcores=_NS)


@functools.partial(
    pl.kernel,
    out_type=jax.ShapeDtypeStruct((_NC, _NPAD), jnp.float32),
    mesh=_mesh,
    scratch_types=[
        pltpu.VMEM((_NCHUNK, _K), jnp.int32),
        pltpu.VMEM((_NCHUNK, _K), jnp.float32),
        pltpu.VMEM_SHARED((_NPAD,), jnp.float32),
        pltpu.SemaphoreType.DMA,
    ],
    compiler_params=pltpu.CompilerParams(use_tc_tiling_on_sc=False),
)
def _sc_degree(col_hbm, ew_hbm, zeros_hbm, out_hbm, col_t, ew_t, deg_sh, sem):
    cid = lax.axis_index("c")
    sid = lax.axis_index("s")
    wid = sid * _NC + cid

    z0 = sid * (_NPAD // _NS)
    pltpu.sync_copy(zeros_hbm.at[pl.ds(z0, _NPAD // _NS)],
                    deg_sh.at[pl.ds(z0, _NPAD // _NS)])
    pltpu.sync_copy(col_hbm.at[wid], col_t)
    pltpu.sync_copy(ew_hbm.at[wid], ew_t)
    plsc.subcore_barrier()

    def body(u, carry):
        cps = [
            pltpu.async_copy(ew_t.at[u * _GRP + j],
                             deg_sh.at[col_t.at[u * _GRP + j]], sem, add=True)
            for j in range(_GRP)
        ]
        for cp in cps:
            cp.wait()
        return carry

    lax.fori_loop(0, _NGRP, body, 0)
    plsc.subcore_barrier()
    pltpu.sync_copy(deg_sh.at[pl.ds(z0, _NPAD // _NS)],
                    out_hbm.at[cid, pl.ds(z0, _NPAD // _NS)])


@functools.partial(
    pl.kernel,
    out_type=jax.ShapeDtypeStruct((_NC, _NPAD, H), jnp.float32),
    mesh=_mesh,
    scratch_types=[
        pltpu.VMEM((_NCHUNK, _K), jnp.int32),
        pltpu.VMEM((_NCHUNK, _K), jnp.int32),
        pltpu.VMEM((_NCHUNK, _K), jnp.float32),
        [pltpu.VMEM((_K, H), jnp.float32) for _ in range(_GRP)],
        pltpu.VMEM_SHARED((_NPAD, H), jnp.float32),
        [pltpu.SemaphoreType.DMA for _ in range(_GRP)],
        [pltpu.SemaphoreType.DMA for _ in range(_GRP)],
    ],
    compiler_params=pltpu.CompilerParams(use_tc_tiling_on_sc=False),
)
def _sc_messages(row_hbm, col_hbm, ew_hbm, hs_hbm, zeros_hbm, out_hbm,
                 row_t, col_t, ew_t, bufs, acc_sh, gsems, ssems):
    cid = lax.axis_index("c")
    sid = lax.axis_index("s")
    wid = sid * _NC + cid
    r0 = sid * _RPT

    pltpu.sync_copy(zeros_hbm.at[pl.ds(r0, _RPT)], acc_sh.at[pl.ds(r0, _RPT)])
    pltpu.sync_copy(row_hbm.at[wid], row_t)
    pltpu.sync_copy(col_hbm.at[wid], col_t)
    pltpu.sync_copy(ew_hbm.at[wid], ew_t)
    plsc.subcore_barrier()

    def _drain_scatter(j):
        pltpu.make_async_copy(bufs[j], acc_sh.at[col_t.at[0]], ssems[j]).wait()

    def _wait_gather(j):
        pltpu.make_async_copy(hs_hbm.at[row_t.at[0]], bufs[j], gsems[j]).wait()

    for j in range(_GRP):
        pltpu.async_copy(hs_hbm.at[row_t.at[j]], bufs[j], gsems[j])

    def body(u, carry):
        i0 = u * _GRP
        for jj in (_GRP - 2, _GRP - 1):
            @pl.when(u > 0)
            def _(jj=jj):
                _drain_scatter(jj)
                pltpu.async_copy(
                    hs_hbm.at[row_t.at[i0 + jj]], bufs[jj], gsems[jj])
        for j in range(_GRP):
            _wait_gather(j)
            buf = bufs[j]
            i = i0 + j
            for j2 in range(_K // 16):
                ew16 = ew_t[i, pl.ds(j2 * 16, 16)]
                for l in range(16):
                    w = ew16[l]
                    e = j2 * 16 + l
                    for f in range(H // 16):
                        sl = pl.ds(f * 16, 16)
                        buf[e, sl] = buf[e, sl] * w
            pltpu.async_copy(buf, acc_sh.at[col_t.at[i]], ssems[j], add=True)
            if j >= 2:
                t = j - 2
                @pl.when(u < _NGRP - 1)
                def _(t=t):
                    _drain_scatter(t)
                    pltpu.async_copy(
                        hs_hbm.at[row_t.at[i0 + _GRP + t]], bufs[t], gsems[t])
        return carry

    lax.fori_loop(0, _NGRP, body, 0)
    for j in range(_GRP):
        pltpu.make_async_copy(bufs[j], acc_sh.at[col_t.at[0]], ssems[j]).wait()
    plsc.subcore_barrier()
    pltpu.sync_copy(acc_sh.at[pl.ds(r0, _RPT)], out_hbm.at[cid, pl.ds(r0, _RPT)])


def _tc_mm_body(x_ref, w_ref, h_ref):
    h_ref[...] = jnp.dot(x_ref[...], w_ref[...],
                         preferred_element_type=jnp.float32)


def _tc_mm(x, w):
    return pl.pallas_call(
        _tc_mm_body,
        out_shape=jax.ShapeDtypeStruct((N, H), jnp.float32),
    )(x, w)


def _tc_scale_body(degp_ref, x_ref, w_ref, hs_ref, dinv_ref):
    deg = 1.0 + degp_ref[0, :N] + degp_ref[1, :N]
    dinv = lax.rsqrt(deg)[:, None]
    h = jnp.dot(x_ref[...], w_ref[...], preferred_element_type=jnp.float32)
    hs_ref[...] = h * dinv
    dinv_ref[...] = dinv


def _tc_scale(deg_p, x, w):
    return pl.pallas_call(
        _tc_scale_body,
        out_shape=(jax.ShapeDtypeStruct((N, H), jnp.float32),
                   jax.ShapeDtypeStruct((N, 1), jnp.float32)),
    )(deg_p, x, w)


def _bn_relu(z, g, be):
    mu = jnp.mean(z, axis=0, keepdims=True)
    var = jnp.mean((z - mu) ** 2, axis=0, keepdims=True)
    zn = (z - mu) * lax.rsqrt(var + EPS) * g + be
    return jnp.maximum(zn, 0.0)


def _tc_mid_body(acc_ref, hs_ref, dinv_ref, b_ref, g_ref, be_ref, w_ref, out_ref):
    dinv = dinv_ref[...]
    z = (acc_ref[0, :N] + hs_ref[...] + acc_ref[1, :N]) * dinv + b_ref[...]
    a = _bn_relu(z, g_ref[...], be_ref[...])
    out_ref[...] = jnp.dot(a, w_ref[...], preferred_element_type=jnp.float32) * dinv


def _tc_mid(acc_p, hs, dinv, b, g, be, w):
    return pl.pallas_call(
        _tc_mid_body,
        out_shape=jax.ShapeDtypeStruct((N, H), jnp.float32),
    )(acc_p, hs, dinv, b, g, be, w)


def _tc_head_body(acc_ref, hs_ref, dinv_ref, b_ref, g_ref, be_ref, batch_ref,
                  wf1_ref, bf1_ref, gf1_ref, bef1_ref, wf2_ref, bf2_ref, out_ref):
    z = (acc_ref[0, :N] + hs_ref[...] + acc_ref[1, :N]) * dinv_ref[...] + b_ref[...]
    h = _bn_relu(z, g_ref[...], be_ref[...])
    onehot = (lax.broadcasted_iota(jnp.int32, (G, N), 0)
              == batch_ref[...][None, :]).astype(jnp.float32)
    s = jnp.dot(onehot, h, preferred_element_type=jnp.float32)
    cnt = jnp.sum(onehot, axis=1, keepdims=True)
    hg = s / jnp.maximum(cnt, 1.0)
    t = jnp.dot(hg, wf1_ref[...], preferred_element_type=jnp.float32) + bf1_ref[...]
    t = _bn_relu(t, gf1_ref[...], bef1_ref[...])
    o = jnp.dot(t, wf2_ref[...], preferred_element_type=jnp.float32) + bf2_ref[...]
    m = jnp.max(o, axis=-1, keepdims=True)
    lse = m + jnp.log(jnp.sum(jnp.exp(o - m), axis=-1, keepdims=True))
    out_ref[...] = o - lse


def _tc_head(acc_p, hs, dinv, b, g, be, batch, wf1, bf1, gf1, bef1, wf2, bf2):
    return pl.pallas_call(
        _tc_head_body,
        out_shape=jax.ShapeDtypeStruct((G, C), jnp.float32),
    )(acc_p, hs, dinv, b, g, be, batch, wf1, bf1, gf1, bef1, wf2, bf2)


def kernel(x, edge_index, batch, edge_attr, pos, W0, b0, g0, be0,
           W1, b1, g1, be1, Wf1, bf1, gf1, bef1, Wf2, bf2):
    del pos
    pad = _EPAD - E
    pad_idx = (jnp.arange(pad, dtype=jnp.int32) % N)
    row = jnp.concatenate([edge_index[0].astype(jnp.int32), pad_idx])
    col = jnp.concatenate([edge_index[1].astype(jnp.int32), pad_idx])
    ew = jnp.concatenate([edge_attr, jnp.zeros((pad,), jnp.float32)])
    row3 = row.reshape(_NW, _NCHUNK, _K)
    col3 = col.reshape(_NW, _NCHUNK, _K)
    ew3 = ew.reshape(_NW, _NCHUNK, _K)
    batch = batch.astype(jnp.int32)
    zeros_n = jnp.zeros((_NPAD,), jnp.float32)
    zeros_nh = jnp.zeros((_NPAD, H), jnp.float32)

    deg_p = _sc_degree(col3, ew3, zeros_n)
    hs0, dinv = _tc_scale(deg_p, x, W0)
    acc0 = _sc_messages(row3, col3, ew3, hs0, zeros_nh)
    hs1 = _tc_mid(acc0, hs0, dinv, b0, g0, be0, W1)
    acc1 = _sc_messages(row3, col3, ew3, hs1, zeros_nh)
    return _tc_head(acc1, hs1, dinv, b1, g1, be1, batch,
                    Wf1, bf1, gf1, bef1, Wf2, bf2)


# speedup vs baseline: 1.0839x; 1.0839x over previous
import functools

import jax
import jax.numpy as jnp
from jax import lax
from jax.experimental import pallas as pl
from jax.experimental.pallas import tpu as pltpu
from jax.experimental.pallas import tpu_sc as plsc

N = 10000
E = 320000
D = 128
H = 64
G = 64
C = 10
EPS = 1e-5

_NC = 2
_NS = 16
_NW = _NC * _NS
_K = 112
_NCHUNK = 90
_GRP = 5
_NGRP = _NCHUNK // _GRP
_EPW = _NCHUNK * _K
_EPAD = _NW * _EPW
_NPAD = 10240
_RPT = _NPAD // _NS

_mesh = plsc.VectorSubcoreMesh(
    core_axis_name="c", subcore_axis_name="s", num_cores=_NC, num_subcores=_NS)


@functools.partial(
    pl.kernel,
    out_type=jax.ShapeDtypeStruct((_NC, _NPAD), jnp.float32),
    mesh=_mesh,
    scratch_types=[
        pltpu.VMEM((_NCHUNK, _K), jnp.int32),
        pltpu.VMEM((_NCHUNK, _K), jnp.float32),
        pltpu.VMEM_SHARED((_NPAD,), jnp.float32),
        pltpu.SemaphoreType.DMA,
    ],
    compiler_params=pltpu.CompilerParams(use_tc_tiling_on_sc=False),
)
def _sc_degree(col_hbm, ew_hbm, zeros_hbm, out_hbm, col_t, ew_t, deg_sh, sem):
    cid = lax.axis_index("c")
    sid = lax.axis_index("s")
    wid = sid * _NC + cid

    z0 = sid * (_NPAD // _NS)
    pltpu.sync_copy(zeros_hbm.at[pl.ds(z0, _NPAD // _NS)],
                    deg_sh.at[pl.ds(z0, _NPAD // _NS)])
    pltpu.sync_copy(col_hbm.at[wid], col_t)
    pltpu.sync_copy(ew_hbm.at[wid], ew_t)
    plsc.subcore_barrier()

    def body(u, carry):
        cps = [
            pltpu.async_copy(ew_t.at[u * _GRP + j],
                             deg_sh.at[col_t.at[u * _GRP + j]], sem, add=True)
            for j in range(_GRP)
        ]
        for cp in cps:
            cp.wait()
        return carry

    lax.fori_loop(0, _NGRP, body, 0)
    plsc.subcore_barrier()
    pltpu.sync_copy(deg_sh.at[pl.ds(z0, _NPAD // _NS)],
                    out_hbm.at[cid, pl.ds(z0, _NPAD // _NS)])


@functools.partial(
    pl.kernel,
    out_type=jax.ShapeDtypeStruct((_NC, _NPAD, H), jnp.float32),
    mesh=_mesh,
    scratch_types=[
        pltpu.VMEM((_NCHUNK, _K), jnp.int32),
        pltpu.VMEM((_NCHUNK, _K), jnp.int32),
        pltpu.VMEM((_NCHUNK, _K), jnp.float32),
        [pltpu.VMEM((_K, H), jnp.float32) for _ in range(_GRP)],
        pltpu.VMEM_SHARED((_NPAD, H), jnp.float32),
        [pltpu.SemaphoreType.DMA for _ in range(_GRP)],
        [pltpu.SemaphoreType.DMA for _ in range(_GRP)],
    ],
    compiler_params=pltpu.CompilerParams(use_tc_tiling_on_sc=False),
)
def _sc_messages(row_hbm, col_hbm, ew_hbm, hs_hbm, zeros_hbm, out_hbm,
                 row_t, col_t, ew_t, bufs, acc_sh, gsems, ssems):
    cid = lax.axis_index("c")
    sid = lax.axis_index("s")
    wid = sid * _NC + cid
    r0 = sid * _RPT

    pltpu.sync_copy(zeros_hbm.at[pl.ds(r0, _RPT)], acc_sh.at[pl.ds(r0, _RPT)])
    pltpu.sync_copy(row_hbm.at[wid], row_t)
    pltpu.sync_copy(col_hbm.at[wid], col_t)
    pltpu.sync_copy(ew_hbm.at[wid], ew_t)
    plsc.subcore_barrier()

    def _drain_scatter(j):
        pltpu.make_async_copy(bufs[j], acc_sh.at[col_t.at[0]], ssems[j]).wait()

    def _wait_gather(j):
        pltpu.make_async_copy(hs_hbm.at[row_t.at[0]], bufs[j], gsems[j]).wait()

    for j in range(_GRP):
        pltpu.async_copy(hs_hbm.at[row_t.at[j]], bufs[j], gsems[j])

    def body(u, carry):
        i0 = u * _GRP
        for jj in (_GRP - 2, _GRP - 1):
            @pl.when(u > 0)
            def _(jj=jj):
                _drain_scatter(jj)
                pltpu.async_copy(
                    hs_hbm.at[row_t.at[i0 + jj]], bufs[jj], gsems[jj])
        for j in range(_GRP):
            _wait_gather(j)
            buf = bufs[j]
            i = i0 + j
            for j2 in range(_K // 16):
                ew16 = ew_t[i, pl.ds(j2 * 16, 16)]
                for l in range(16):
                    w = ew16[l]
                    e = j2 * 16 + l
                    for f in range(H // 16):
                        sl = pl.ds(f * 16, 16)
                        buf[e, sl] = buf[e, sl] * w
            pltpu.async_copy(buf, acc_sh.at[col_t.at[i]], ssems[j], add=True)
            if j >= 2:
                t = j - 2
                @pl.when(u < _NGRP - 1)
                def _(t=t):
                    _drain_scatter(t)
                    pltpu.async_copy(
                        hs_hbm.at[row_t.at[i0 + _GRP + t]], bufs[t], gsems[t])
        return carry

    lax.fori_loop(0, _NGRP, body, 0)
    for j in range(_GRP):
        pltpu.make_async_copy(bufs[j], acc_sh.at[col_t.at[0]], ssems[j]).wait()
    plsc.subcore_barrier()
    pltpu.sync_copy(acc_sh.at[pl.ds(r0, _RPT)], out_hbm.at[cid, pl.ds(r0, _RPT)])


def _tc_scale_body(degp_ref, x_ref, w_ref, hs_ref, dinv_ref):
    deg = 1.0 + degp_ref[0, :N] + degp_ref[1, :N]
    dinv = lax.rsqrt(deg)[:, None]
    h = jnp.dot(x_ref[...], w_ref[...], preferred_element_type=jnp.float32)
    hs_ref[...] = h * dinv
    dinv_ref[...] = dinv


def _tc_scale(deg_p, x, w):
    return pl.pallas_call(
        _tc_scale_body,
        out_shape=(jax.ShapeDtypeStruct((N, H), jnp.float32),
                   jax.ShapeDtypeStruct((N, 1), jnp.float32)),
    )(deg_p, x, w)


def _bn_relu(z, g, be):
    mu = jnp.mean(z, axis=0, keepdims=True)
    var = jnp.mean((z - mu) ** 2, axis=0, keepdims=True)
    zn = (z - mu) * lax.rsqrt(var + EPS) * g + be
    return jnp.maximum(zn, 0.0)


def _tc_mid_body(acc_ref, hs_ref, dinv_ref, b_ref, g_ref, be_ref, w_ref, out_ref):
    dinv = dinv_ref[...]
    z = (acc_ref[0, :N] + hs_ref[...] + acc_ref[1, :N]) * dinv + b_ref[...]
    a = _bn_relu(z, g_ref[...], be_ref[...])
    out_ref[...] = jnp.dot(a, w_ref[...], preferred_element_type=jnp.float32) * dinv


def _tc_mid(acc_p, hs, dinv, b, g, be, w):
    return pl.pallas_call(
        _tc_mid_body,
        out_shape=jax.ShapeDtypeStruct((N, H), jnp.float32),
    )(acc_p, hs, dinv, b, g, be, w)


def _tc_head_body(acc_ref, hs_ref, dinv_ref, b_ref, g_ref, be_ref, batch_ref,
                  wf1_ref, bf1_ref, gf1_ref, bef1_ref, wf2_ref, bf2_ref, out_ref):
    z = (acc_ref[0, :N] + hs_ref[...] + acc_ref[1, :N]) * dinv_ref[...] + b_ref[...]
    h = _bn_relu(z, g_ref[...], be_ref[...])
    onehot = (lax.broadcasted_iota(jnp.int32, (G, N), 0)
              == batch_ref[...][None, :]).astype(jnp.float32)
    s = jnp.dot(onehot, h, preferred_element_type=jnp.float32)
    cnt = jnp.sum(onehot, axis=1, keepdims=True)
    hg = s / jnp.maximum(cnt, 1.0)
    t = jnp.dot(hg, wf1_ref[...], preferred_element_type=jnp.float32) + bf1_ref[...]
    t = _bn_relu(t, gf1_ref[...], bef1_ref[...])
    o = jnp.dot(t, wf2_ref[...], preferred_element_type=jnp.float32) + bf2_ref[...]
    m = jnp.max(o, axis=-1, keepdims=True)
    lse = m + jnp.log(jnp.sum(jnp.exp(o - m), axis=-1, keepdims=True))
    out_ref[...] = o - lse


def _tc_head(acc_p, hs, dinv, b, g, be, batch, wf1, bf1, gf1, bef1, wf2, bf2):
    return pl.pallas_call(
        _tc_head_body,
        out_shape=jax.ShapeDtypeStruct((G, C), jnp.float32),
    )(acc_p, hs, dinv, b, g, be, batch, wf1, bf1, gf1, bef1, wf2, bf2)


def kernel(x, edge_index, batch, edge_attr, pos, W0, b0, g0, be0,
           W1, b1, g1, be1, Wf1, bf1, gf1, bef1, Wf2, bf2):
    del pos
    pad = _EPAD - E
    pad_idx = (jnp.arange(pad, dtype=jnp.int32) % N)
    row = jnp.concatenate([edge_index[0].astype(jnp.int32), pad_idx])
    col = jnp.concatenate([edge_index[1].astype(jnp.int32), pad_idx])
    ew = jnp.concatenate([edge_attr, jnp.zeros((pad,), jnp.float32)])
    row3 = row.reshape(_NW, _NCHUNK, _K)
    col3 = col.reshape(_NW, _NCHUNK, _K)
    ew3 = ew.reshape(_NW, _NCHUNK, _K)
    batch = batch.astype(jnp.int32)
    zeros_n = jnp.zeros((_NPAD,), jnp.float32)
    zeros_nh = jnp.zeros((_NPAD, H), jnp.float32)

    deg_p = _sc_degree(col3, ew3, zeros_n)
    hs0, dinv = _tc_scale(deg_p, x, W0)
    acc0 = _sc_messages(row3, col3, ew3, hs0, zeros_nh)
    hs1 = _tc_mid(acc0, hs0, dinv, b0, g0, be0, W1)
    acc1 = _sc_messages(row3, col3, ew3, hs1, zeros_nh)
    return _tc_head(acc1, hs1, dinv, b1, g1, be1, batch,
                    Wf1, bf1, gf1, bef1, Wf2, bf2)
